# baseline (speedup 1.0000x reference)
import functools

import jax
import jax.numpy as jnp
from jax import lax
from jax.experimental import pallas as pl
from jax.experimental.pallas import tpu as pltpu
from jax.experimental.pallas import tpu_sc as plsc

_ROWS_T = 200
_COLS_T = 16384
_NC = 2
_NS = 16
_NW = _NC * _NS
_CW = _COLS_T // _NW
_LANES = 16
_CHUNK_R = 40
_N_CHUNKS = _ROWS_T // _CHUNK_R
_GPR = _CW // _LANES


_N_TABLES = 2
_TBL = 128
_NCHAIN = _N_TABLES * _LANES


def _gather_sum_body(zt_hbm, w_hbm, out_hbm, w_v, cnt_v, z0_v, z1_v, acc_v,
                     sem0, sem1):
    wid = lax.axis_index("s") * _NC + lax.axis_index("c")
    base = wid * _CW

    zero = jnp.zeros((_LANES,), jnp.float32)
    ones = jnp.ones((_LANES,), jnp.float32)
    w_v[pl.ds(_TBL - _LANES, _LANES)] = zero
    pltpu.sync_copy(w_hbm, w_v.at[pl.ds(0, 119)])

    for i in range(_N_TABLES * _TBL // _LANES):
        cnt_v[pl.ds(i * _LANES, _LANES)] = zero

    bufs = (z0_v, z1_v)
    sems = (sem0, sem1)
    copies = [None, None]
    copies[0] = pltpu.async_copy(
        zt_hbm.at[pl.ds(0, _CHUNK_R), pl.ds(base, _CW)], z0_v, sem0)

    for s in range(_N_CHUNKS):
        if s + 1 < _N_CHUNKS:
            copies[(s + 1) % 2] = pltpu.async_copy(
                zt_hbm.at[pl.ds((s + 1) * _CHUNK_R, _CHUNK_R),
                          pl.ds(base, _CW)],
                bufs[(s + 1) % 2], sems[(s + 1) % 2])
        copies[s % 2].wait()
        z_v = bufs[s % 2]

        @plsc.parallel_loop(0, _CHUNK_R, step=1, unroll=2)
        def body(r):
            for k in range(_GPR):
                idx = z_v[r, pl.ds(k * _LANES, _LANES)]
                plsc.addupdate_scatter(
                    cnt_v, [idx + (k % _N_TABLES) * _TBL], ones)

    acc = zero
    for g in range(_TBL // _LANES):
        c = cnt_v[pl.ds(g * _LANES, _LANES)]
        for t in range(1, _N_TABLES):
            c = c + cnt_v[pl.ds(t * _TBL + g * _LANES, _LANES)]
        acc = acc + c * w_v[pl.ds(g * _LANES, _LANES)]
    acc_v[...] = acc
    pltpu.sync_copy(acc_v, out_hbm.at[wid])


@jax.jit
def _gather_sum(zt, w_pad):
    mesh = plsc.VectorSubcoreMesh(core_axis_name="c", subcore_axis_name="s")
    run = functools.partial(
        pl.kernel,
        mesh=mesh,
        compiler_params=pltpu.CompilerParams(needs_layout_passes=False),
        out_type=jax.ShapeDtypeStruct((_NW, _LANES), jnp.float32),
        scratch_types=[
            pltpu.VMEM((_TBL,), jnp.float32),
            pltpu.VMEM((_N_TABLES * _TBL,), jnp.float32),
            pltpu.VMEM((_CHUNK_R, _CW), jnp.int32),
            pltpu.VMEM((_CHUNK_R, _CW), jnp.int32),
            pltpu.VMEM((_LANES,), jnp.float32),
            pltpu.SemaphoreType.DMA,
            pltpu.SemaphoreType.DMA,
        ],
    )(_gather_sum_body)
    return run(zt, w_pad)


def kernel(Z, ref_weight):
    partials = _gather_sum(Z.T, ref_weight.reshape(-1))
    return partials.sum()


# speedup vs baseline: 1.1779x; 1.1779x over previous
import functools

import jax
import jax.numpy as jnp
from jax import lax
from jax.experimental import pallas as pl
from jax.experimental.pallas import tpu as pltpu
from jax.experimental.pallas import tpu_sc as plsc

_ROWS_T = 200
_COLS_T = 16384
_NC = 2
_NS = 16
_NW = _NC * _NS
_CW = _COLS_T // _NW
_LANES = 16
_CHUNK = 128
_N_CHUNKS = _CW // _CHUNK
_GPR = _CHUNK // _LANES


_N_TABLES = 2
_TBL = 128
_NCHAIN = _N_TABLES * _LANES


def _gather_sum_body(zt_hbm, w_hbm, out_hbm, w_v, cnt_a, cnt_b, z0_v, z1_v,
                     acc_v, sem0, sem1):
    wid = lax.axis_index("s") * _NC + lax.axis_index("c")
    base = wid * _CW

    zero = jnp.zeros((_LANES,), jnp.float32)
    ones = jnp.ones((_LANES,), jnp.float32)
    w_v[pl.ds(_TBL - _LANES, _LANES)] = zero
    pltpu.sync_copy(w_hbm, w_v.at[pl.ds(0, 119)])

    for i in range(_TBL // _LANES):
        cnt_a[pl.ds(i * _LANES, _LANES)] = zero
        cnt_b[pl.ds(i * _LANES, _LANES)] = zero

    bufs = (z0_v, z1_v)
    sems = (sem0, sem1)
    copies = [None, None]
    copies[0] = pltpu.async_copy(
        zt_hbm.at[:, pl.ds(base, _CHUNK)], z0_v, sem0)

    for s in range(_N_CHUNKS):
        if s + 1 < _N_CHUNKS:
            copies[(s + 1) % 2] = pltpu.async_copy(
                zt_hbm.at[:, pl.ds(base + (s + 1) * _CHUNK, _CHUNK)],
                bufs[(s + 1) % 2], sems[(s + 1) % 2])
        copies[s % 2].wait()
        z_v = bufs[s % 2]

        @plsc.parallel_loop(0, _ROWS_T, step=1, unroll=4)
        def body(r):
            for k in range(_GPR):
                idx = z_v[r, pl.ds(k * _LANES, _LANES)]
                plsc.addupdate_scatter(
                    cnt_a if k % 2 == 0 else cnt_b, [idx], ones)

    acc = zero
    for g in range(_TBL // _LANES):
        c = cnt_a[pl.ds(g * _LANES, _LANES)] + cnt_b[pl.ds(g * _LANES, _LANES)]
        acc = acc + c * w_v[pl.ds(g * _LANES, _LANES)]
    acc_v[...] = acc
    pltpu.sync_copy(acc_v, out_hbm.at[wid])


@jax.jit
def _gather_sum(zt, w_pad):
    mesh = plsc.VectorSubcoreMesh(core_axis_name="c", subcore_axis_name="s")
    run = functools.partial(
        pl.kernel,
        mesh=mesh,
        compiler_params=pltpu.CompilerParams(needs_layout_passes=False),
        out_type=jax.ShapeDtypeStruct((_NW, _LANES), jnp.float32),
        scratch_types=[
            pltpu.VMEM((_TBL,), jnp.float32),
            pltpu.VMEM((_TBL,), jnp.float32),
            pltpu.VMEM((_TBL,), jnp.float32),
            pltpu.VMEM((_ROWS_T, _CHUNK), jnp.int32),
            pltpu.VMEM((_ROWS_T, _CHUNK), jnp.int32),
            pltpu.VMEM((_LANES,), jnp.float32),
            pltpu.SemaphoreType.DMA,
            pltpu.SemaphoreType.DMA,
        ],
    )(_gather_sum_body)
    return run(zt, w_pad)


def kernel(Z, ref_weight):
    partials = _gather_sum(Z.T, ref_weight.reshape(-1))
    return partials.sum()
